# baseline (speedup 1.0000x reference)
import functools

import jax
import jax.numpy as jnp
from jax.experimental import pallas as pl


def _mlp3_body(g_ref, w0_ref, b0_ref, w1_ref, b1_ref, w2_ref, b2_ref, out_ref):
    h = jnp.dot(g_ref[...], w0_ref[...], preferred_element_type=jnp.float32)
    h = jnp.maximum(h + b0_ref[...], 0.0)
    h = jnp.dot(h, w1_ref[...], preferred_element_type=jnp.float32)
    h = jnp.maximum(h + b1_ref[...], 0.0)
    h = jnp.dot(h, w2_ref[...], preferred_element_type=jnp.float32)
    out_ref[...] = h + b2_ref[...]


@functools.partial(jax.jit, static_argnames=("chunk",))
def _mlp3(g, w0, b0, w1, b1, w2, b2, chunk=2048):
    n, f0 = g.shape
    f1 = w1.shape[0]
    f2 = w2.shape[0]
    f3 = w2.shape[1]
    n_pad = (n + chunk - 1) // chunk * chunk
    if n_pad != n:
        g = jnp.pad(g, ((0, n_pad - n), (0, 0)))
    grid = n_pad // chunk
    out = pl.pallas_call(
        _mlp3_body,
        grid=(grid,),
        in_specs=[
            pl.BlockSpec((chunk, f0), lambda i: (i, 0)),
            pl.BlockSpec((f0, f1), lambda i: (0, 0)),
            pl.BlockSpec((1, f1), lambda i: (0, 0)),
            pl.BlockSpec((f1, f2), lambda i: (0, 0)),
            pl.BlockSpec((1, f2), lambda i: (0, 0)),
            pl.BlockSpec((f2, f3), lambda i: (0, 0)),
            pl.BlockSpec((1, f3), lambda i: (0, 0)),
        ],
        out_specs=pl.BlockSpec((chunk, f3), lambda i: (i, 0)),
        out_shape=jax.ShapeDtypeStruct((n_pad, f3), jnp.float32),
    )(g, w0, b0.reshape(1, -1), w1, b1.reshape(1, -1), w2, b2.reshape(1, -1))
    return out[:n]


def _point_conv(x, pos, edge_index, p, pre, add_self_loops, num_nodes):
    src = edge_index[0]
    dst = edge_index[1]
    if add_self_loops:
        loop = jnp.arange(num_nodes, dtype=src.dtype)
        src = jnp.concatenate([src, loop])
        dst = jnp.concatenate([dst, loop])
    msg = jnp.concatenate([x[src], pos[src] - pos[dst]], axis=1)
    m = _mlp3(msg, p[pre + '_W0'], p[pre + '_b0'], p[pre + '_W1'],
              p[pre + '_b1'], p[pre + '_W2'], p[pre + '_b2'])
    agg = jax.ops.segment_max(m, dst, num_segments=num_nodes)
    return jnp.where(jnp.isfinite(agg), agg, 0.0)


def _bn(h, g, b):
    m = h.mean(0)
    v = h.var(0)
    return (h - m) / jnp.sqrt(v + 1e-05) * g + b


def kernel(x, pos, params, edge_index, batch, pool_perm1, edge_index2, pool_perm2, edge_index3):
    N = x.shape[0]
    h = _point_conv(x, pos, edge_index, params, 'b1', True, N)
    h = h[pool_perm1]
    pos2 = pos[pool_perm1]
    batch2 = batch[pool_perm1]
    h = _point_conv(h, pos2, edge_index2, params, 'b2', False, pool_perm1.shape[0])
    h = h[pool_perm2]
    pos3 = pos2[pool_perm2]
    batch3 = batch2[pool_perm2]
    h = _point_conv(h, pos3, edge_index3, params, 'b3', False, pool_perm2.shape[0])
    g = jax.ops.segment_max(h, batch3, num_segments=16)
    g = jnp.where(jnp.isfinite(g), g, 0.0)
    out = jax.nn.relu(_bn(g, params['bn1_g'], params['bn1_b']))
    out = out @ params['m_W1'] + params['m_b1']
    out = jax.nn.relu(_bn(out, params['bn2_g'], params['bn2_b']))
    out = out @ params['m_W2'] + params['m_b2']
    out = jax.nn.relu(_bn(out, params['bn3_g'], params['bn3_b']))
    out = out @ params['m_W3'] + params['m_b3']
    return out


# speedup vs baseline: 1.1588x; 1.1588x over previous
import functools

import jax
import jax.numpy as jnp
from jax.experimental import pallas as pl


def _mlp3_body(g_ref, w0_ref, b0_ref, w1_ref, b1_ref, w2_ref, b2_ref, out_ref):
    h = jnp.dot(g_ref[...], w0_ref[...], preferred_element_type=jnp.float32)
    h = jnp.maximum(h + b0_ref[...], 0.0)
    h = jnp.dot(h, w1_ref[...], preferred_element_type=jnp.float32)
    h = jnp.maximum(h + b1_ref[...], 0.0)
    h = jnp.dot(h, w2_ref[...], preferred_element_type=jnp.float32)
    out_ref[...] = h + b2_ref[...]


@functools.partial(jax.jit, static_argnames=("chunk",))
def _mlp3(g, w0, b0, w1, b1, w2, b2, chunk=2048):
    n, f0 = g.shape
    f1 = w1.shape[0]
    f2 = w2.shape[0]
    f3 = w2.shape[1]
    n_pad = (n + chunk - 1) // chunk * chunk
    if n_pad != n:
        g = jnp.pad(g, ((0, n_pad - n), (0, 0)))
    grid = n_pad // chunk
    out = pl.pallas_call(
        _mlp3_body,
        grid=(grid,),
        in_specs=[
            pl.BlockSpec((chunk, f0), lambda i: (i, 0)),
            pl.BlockSpec((f0, f1), lambda i: (0, 0)),
            pl.BlockSpec((1, f1), lambda i: (0, 0)),
            pl.BlockSpec((f1, f2), lambda i: (0, 0)),
            pl.BlockSpec((1, f2), lambda i: (0, 0)),
            pl.BlockSpec((f2, f3), lambda i: (0, 0)),
            pl.BlockSpec((1, f3), lambda i: (0, 0)),
        ],
        out_specs=pl.BlockSpec((chunk, f3), lambda i: (i, 0)),
        out_shape=jax.ShapeDtypeStruct((n_pad, f3), jnp.float32),
    )(g, w0, b0.reshape(1, -1), w1, b1.reshape(1, -1), w2, b2.reshape(1, -1))
    return out[:n]


def _point_conv(x, pos, edge_index, p, pre, add_self_loops, num_nodes,
                n_chunks=4):
    src = edge_index[0]
    dst = edge_index[1]
    if add_self_loops:
        loop = jnp.arange(num_nodes, dtype=src.dtype)
        src = jnp.concatenate([src, loop])
        dst = jnp.concatenate([dst, loop])
    E = src.shape[0]
    bounds = [E * i // n_chunks for i in range(n_chunks + 1)]
    agg = None
    for i in range(n_chunks):
        lo, hi = bounds[i], bounds[i + 1]
        s, dvec = src[lo:hi], dst[lo:hi]
        msg = jnp.concatenate([x[s], pos[s] - pos[dvec]], axis=1)
        m = _mlp3(msg, p[pre + '_W0'], p[pre + '_b0'], p[pre + '_W1'],
                  p[pre + '_b1'], p[pre + '_W2'], p[pre + '_b2'])
        part = jax.ops.segment_max(m, dvec, num_segments=num_nodes)
        agg = part if agg is None else jnp.maximum(agg, part)
    return jnp.where(jnp.isfinite(agg), agg, 0.0)


def _bn(h, g, b):
    m = h.mean(0)
    v = h.var(0)
    return (h - m) / jnp.sqrt(v + 1e-05) * g + b


def kernel(x, pos, params, edge_index, batch, pool_perm1, edge_index2, pool_perm2, edge_index3):
    N = x.shape[0]
    h = _point_conv(x, pos, edge_index, params, 'b1', True, N)
    h = h[pool_perm1]
    pos2 = pos[pool_perm1]
    batch2 = batch[pool_perm1]
    h = _point_conv(h, pos2, edge_index2, params, 'b2', False, pool_perm1.shape[0])
    h = h[pool_perm2]
    pos3 = pos2[pool_perm2]
    batch3 = batch2[pool_perm2]
    h = _point_conv(h, pos3, edge_index3, params, 'b3', False, pool_perm2.shape[0])
    g = jax.ops.segment_max(h, batch3, num_segments=16)
    g = jnp.where(jnp.isfinite(g), g, 0.0)
    out = jax.nn.relu(_bn(g, params['bn1_g'], params['bn1_b']))
    out = out @ params['m_W1'] + params['m_b1']
    out = jax.nn.relu(_bn(out, params['bn2_g'], params['bn2_b']))
    out = out @ params['m_W2'] + params['m_b2']
    out = jax.nn.relu(_bn(out, params['bn3_g'], params['bn3_b']))
    out = out @ params['m_W3'] + params['m_b3']
    return out


# speedup vs baseline: 1.1597x; 1.0008x over previous
import functools

import jax
import jax.numpy as jnp
from jax.experimental import pallas as pl


def _mlp3_body(g_ref, w0_ref, b0_ref, w1_ref, b1_ref, w2_ref, b2_ref, out_ref):
    h = jnp.dot(g_ref[...], w0_ref[...], preferred_element_type=jnp.float32)
    h = jnp.maximum(h + b0_ref[...], 0.0)
    h = jnp.dot(h, w1_ref[...], preferred_element_type=jnp.float32)
    h = jnp.maximum(h + b1_ref[...], 0.0)
    h = jnp.dot(h, w2_ref[...], preferred_element_type=jnp.float32)
    out_ref[...] = h + b2_ref[...]


@functools.partial(jax.jit, static_argnames=("chunk",))
def _mlp3(g, w0, b0, w1, b1, w2, b2, chunk=2048):
    n, f0 = g.shape
    f1 = w1.shape[0]
    f2 = w2.shape[0]
    f3 = w2.shape[1]
    n_pad = (n + chunk - 1) // chunk * chunk
    if n_pad != n:
        g = jnp.pad(g, ((0, n_pad - n), (0, 0)))
    grid = n_pad // chunk
    out = pl.pallas_call(
        _mlp3_body,
        grid=(grid,),
        in_specs=[
            pl.BlockSpec((chunk, f0), lambda i: (i, 0)),
            pl.BlockSpec((f0, f1), lambda i: (0, 0)),
            pl.BlockSpec((1, f1), lambda i: (0, 0)),
            pl.BlockSpec((f1, f2), lambda i: (0, 0)),
            pl.BlockSpec((1, f2), lambda i: (0, 0)),
            pl.BlockSpec((f2, f3), lambda i: (0, 0)),
            pl.BlockSpec((1, f3), lambda i: (0, 0)),
        ],
        out_specs=pl.BlockSpec((chunk, f3), lambda i: (i, 0)),
        out_shape=jax.ShapeDtypeStruct((n_pad, f3), jnp.float32),
    )(g, w0, b0.reshape(1, -1), w1, b1.reshape(1, -1), w2, b2.reshape(1, -1))
    return out[:n]


def _point_conv(x, pos, edge_index, p, pre, add_self_loops, num_nodes,
                n_chunks=8):
    src = edge_index[0]
    dst = edge_index[1]
    if add_self_loops:
        loop = jnp.arange(num_nodes, dtype=src.dtype)
        src = jnp.concatenate([src, loop])
        dst = jnp.concatenate([dst, loop])
    E = src.shape[0]
    bounds = [E * i // n_chunks for i in range(n_chunks + 1)]
    agg = None
    for i in range(n_chunks):
        lo, hi = bounds[i], bounds[i + 1]
        s, dvec = src[lo:hi], dst[lo:hi]
        msg = jnp.concatenate([x[s], pos[s] - pos[dvec]], axis=1)
        m = _mlp3(msg, p[pre + '_W0'], p[pre + '_b0'], p[pre + '_W1'],
                  p[pre + '_b1'], p[pre + '_W2'], p[pre + '_b2'])
        part = jax.ops.segment_max(m, dvec, num_segments=num_nodes)
        agg = part if agg is None else jnp.maximum(agg, part)
    return jnp.where(jnp.isfinite(agg), agg, 0.0)


def _bn(h, g, b):
    m = h.mean(0)
    v = h.var(0)
    return (h - m) / jnp.sqrt(v + 1e-05) * g + b


def kernel(x, pos, params, edge_index, batch, pool_perm1, edge_index2, pool_perm2, edge_index3):
    N = x.shape[0]
    h = _point_conv(x, pos, edge_index, params, 'b1', True, N)
    h = h[pool_perm1]
    pos2 = pos[pool_perm1]
    batch2 = batch[pool_perm1]
    h = _point_conv(h, pos2, edge_index2, params, 'b2', False, pool_perm1.shape[0])
    h = h[pool_perm2]
    pos3 = pos2[pool_perm2]
    batch3 = batch2[pool_perm2]
    h = _point_conv(h, pos3, edge_index3, params, 'b3', False, pool_perm2.shape[0])
    g = jax.ops.segment_max(h, batch3, num_segments=16)
    g = jnp.where(jnp.isfinite(g), g, 0.0)
    out = jax.nn.relu(_bn(g, params['bn1_g'], params['bn1_b']))
    out = out @ params['m_W1'] + params['m_b1']
    out = jax.nn.relu(_bn(out, params['bn2_g'], params['bn2_b']))
    out = out @ params['m_W2'] + params['m_b2']
    out = jax.nn.relu(_bn(out, params['bn3_g'], params['bn3_b']))
    out = out @ params['m_W3'] + params['m_b3']
    return out
